# baseline (speedup 1.0000x reference)
import functools

import jax
import jax.numpy as jnp
from jax import lax
from jax.experimental import pallas as pl
from jax.experimental.pallas import tpu as pltpu
from jax.experimental.pallas import tpu_sc as plsc

_B = 16384
_A = 64
_D = 128

_IDX_CHUNK = 128


@functools.cache
def _make_sc_gather():
    info = plsc.get_sparse_core_info()
    nw = info.num_cores * info.num_subcores
    b_per_w = _B // nw
    chunks = b_per_w // _IDX_CHUNK
    n_chunks_total = _B // _IDX_CHUNK
    mesh = plsc.VectorSubcoreMesh(core_axis_name="c", subcore_axis_name="s")

    @functools.partial(
        pl.kernel,
        mesh=mesh,
        out_type=jax.ShapeDtypeStruct((n_chunks_total, _IDX_CHUNK, _D),
                                      jnp.float32),
        scratch_types=[
            pltpu.VMEM((chunks, _IDX_CHUNK), jnp.int32),
            pltpu.VMEM((chunks, _IDX_CHUNK, _D), jnp.float32),
            pltpu.SemaphoreType.DMA,
            pltpu.SemaphoreType.DMA,
        ],
    )
    def gather(table_hbm, idx_hbm, out_hbm, idx_v, rows_v, sem_in, sem_out):
        wid = lax.axis_index("s") * info.num_cores + lax.axis_index("c")
        pltpu.sync_copy(idx_hbm.at[pl.ds(wid * chunks, chunks)], idx_v)
        copies = [
            pltpu.async_copy(table_hbm.at[idx_v.at[j]], rows_v.at[j], sem_in)
            for j in range(chunks)
        ]
        outs = []
        for j, c in enumerate(copies):
            c.wait()
            outs.append(
                pltpu.async_copy(rows_v.at[j],
                                 out_hbm.at[wid * chunks + j], sem_out))
        for o in outs:
            o.wait()

    return gather


_PBLK = 256
_NBUF = 6
_NSTEP = _B // _PBLK


def _expand_manual_body(bw_ref, asp_ref, out_ref, vbuf, sems):
    i = pl.program_id(0)

    def _issue_wait(step):
        b = jax.lax.rem(step, _NBUF)
        pltpu.make_async_copy(
            vbuf.at[b],
            out_ref.at[pl.ds(step * _PBLK, _PBLK)],
            sems.at[b],
        ).wait()

    b = jax.lax.rem(i, _NBUF)

    @pl.when(i >= _NBUF)
    def _():
        _issue_wait(i - _NBUF)

    asp = asp_ref[...]
    bw = bw_ref[:, :_A]
    vbuf[b] = bw[:, :, None] * asp[None]
    pltpu.async_copy(vbuf.at[b], out_ref.at[pl.ds(i * _PBLK, _PBLK)],
                     sems.at[b])

    @pl.when(i == _NSTEP - 1)
    def _():
        for d in range(_NBUF):
            _issue_wait(_NSTEP - _NBUF + d)


def _expand_manual(bw_wide, aspects):
    return pl.pallas_call(
        _expand_manual_body,
        grid=(_NSTEP,),
        in_specs=[
            pl.BlockSpec((_PBLK, _D), lambda i: (i, 0)),
            pl.BlockSpec((_A, _D), lambda i: (0, 0)),
        ],
        out_specs=pl.BlockSpec(memory_space=pl.ANY),
        out_shape=jax.ShapeDtypeStruct((_B, _A, _D), jnp.float32),
        scratch_shapes=[
            pltpu.VMEM((_NBUF, _PBLK, _A, _D), jnp.float32),
            pltpu.SemaphoreType.DMA((_NBUF,)),
        ],
        compiler_params=pltpu.CompilerParams(
            vmem_limit_bytes=100 * 1024 * 1024),
    )(bw_wide, aspects)


def kernel(brand_list, brand_table, aspects):
    idx = brand_list.astype(jnp.int32).reshape(_B // _IDX_CHUNK, _IDX_CHUNK)
    table_wide = jnp.pad(brand_table, ((0, 0), (0, _D - _A)))
    bw3 = _make_sc_gather()(table_wide, idx)
    bw_wide = bw3.reshape(_B, _D)
    return _expand_manual(bw_wide, aspects)
